# initial kernel scaffold (unmeasured)
import jax
import jax.numpy as jnp
from jax import lax
from jax.experimental import pallas as pl
from jax.experimental.pallas import tpu as pltpu


def kernel(
    x,
):
    def body(*refs):
        pass

    out_shape = jax.ShapeDtypeStruct(..., jnp.float32)
    return pl.pallas_call(body, out_shape=out_shape)(...)


# baseline (device time: 17518 ns/iter reference)
import jax
import jax.numpy as jnp
from jax import lax
from jax.experimental import pallas as pl
from jax.experimental.pallas import tpu as pltpu

N_DEV = 8
MASKS = (1, 3, 4)


def kernel(x):
    m_per, n = x.shape

    def body(x_ref, out_ref, comm_ref, send_sems, recv_sems):
        my = lax.axis_index("i")

        out_ref[...] = jnp.sum(x_ref[...], axis=0, keepdims=True)

        for r, mask in enumerate(MASKS):
            partner = jnp.bitwise_xor(my, mask)
            rdma = pltpu.make_async_remote_copy(
                src_ref=out_ref,
                dst_ref=comm_ref.at[r],
                send_sem=send_sems.at[r],
                recv_sem=recv_sems.at[r],
                device_id=(partner,),
                device_id_type=pl.DeviceIdType.MESH,
            )
            rdma.start()
            rdma.wait()
            out_ref[...] = out_ref[...] + comm_ref[r]

    return pl.pallas_call(
        body,
        out_shape=jax.ShapeDtypeStruct((1, n), x.dtype),
        in_specs=[pl.BlockSpec(memory_space=pltpu.VMEM)],
        out_specs=pl.BlockSpec(memory_space=pltpu.VMEM),
        scratch_shapes=[
            pltpu.VMEM((len(MASKS), 1, n), x.dtype),
            pltpu.SemaphoreType.DMA((len(MASKS),)),
            pltpu.SemaphoreType.DMA((len(MASKS),)),
        ],
    )(x)


# device time: 14911 ns/iter; 1.1748x vs baseline; 1.1748x over previous
import jax
import jax.numpy as jnp
from jax import lax
from jax.experimental import pallas as pl
from jax.experimental.pallas import tpu as pltpu

N_DEV = 8


def kernel(x):
    m_per, n = x.shape

    def body(x_ref, out_ref, comm_ref, send_sems, recv_sems):
        my = lax.axis_index("i")

        out_ref[...] = jnp.sum(x_ref[...], axis=0, keepdims=True)

        rdmas = []
        for j in range(1, N_DEV):
            dst = lax.rem(my + j, N_DEV)
            rdma = pltpu.make_async_remote_copy(
                src_ref=out_ref,
                dst_ref=comm_ref.at[N_DEV - 1 - j],
                send_sem=send_sems.at[j - 1],
                recv_sem=recv_sems.at[N_DEV - 1 - j],
                device_id=(dst,),
                device_id_type=pl.DeviceIdType.MESH,
            )
            rdma.start()
            rdmas.append(rdma)
        for rdma in rdmas:
            rdma.wait()

        out_ref[...] = out_ref[...] + jnp.sum(comm_ref[...], axis=0)

    return pl.pallas_call(
        body,
        out_shape=jax.ShapeDtypeStruct((1, n), x.dtype),
        in_specs=[pl.BlockSpec(memory_space=pltpu.VMEM)],
        out_specs=pl.BlockSpec(memory_space=pltpu.VMEM),
        scratch_shapes=[
            pltpu.VMEM((N_DEV - 1, 1, n), x.dtype),
            pltpu.SemaphoreType.DMA((N_DEV - 1,)),
            pltpu.SemaphoreType.DMA((N_DEV - 1,)),
        ],
    )(x)
